# baseline (speedup 1.0000x reference)
import functools

import jax
import jax.numpy as jnp
from jax import lax
from jax.experimental import pallas as pl
from jax.experimental.pallas import tpu as pltpu
from jax.experimental.pallas import tpu_sc as plsc

N = 10000
E = 320000
D = 128
NC = 2
NS = 16
TILES = NC * NS
NPAD = 10240
CHUNK = 128
CPT = 80
EPAD = TILES * CPT * CHUNK
RPT = NPAD // NS
NBUF = 2
CPAD = CPT + NBUF

_MESH = plsc.VectorSubcoreMesh(
    core_axis_name="c", subcore_axis_name="s", num_cores=NC, num_subcores=NS
)


_BROWS = 64


@functools.partial(
    pl.kernel,
    out_type=jax.ShapeDtypeStruct((NC, NPAD, D), jnp.float32),
    mesh=_MESH,
    scratch_types=[
        pltpu.VMEM((2, CHUNK), jnp.int32),
        pltpu.VMEM((CHUNK, D), jnp.float32),
        pltpu.VMEM((_BROWS, D), jnp.float32),
        pltpu.VMEM_SHARED((NPAD, D), jnp.float32),
    ],
)
def _deg_kernel(idx_hbm, out_hbm, idx0, ones_v, bounce, acc):
    c = lax.axis_index("c")
    s = lax.axis_index("s")
    wid = c * NS + s

    ones16 = jnp.full((16,), 1.0, jnp.float32)
    zero16 = jnp.zeros((16,), jnp.float32)

    def _fill(i, carry):
        ones_v[i // 8, pl.ds((i % 8) * 16, 16)] = ones16
        return carry

    lax.fori_loop(0, CHUNK * 8, _fill, 0)

    def _zero(i, carry):
        bounce[i // 8, pl.ds((i % 8) * 16, 16)] = zero16
        return carry

    lax.fori_loop(0, _BROWS * 8, _zero, 0)

    t0 = s * RPT

    def _zacc(b, carry):
        pltpu.sync_copy(bounce, acc.at[pl.ds(t0 + b * _BROWS, _BROWS)])
        return carry

    lax.fori_loop(0, RPT // _BROWS, _zacc, 0)
    plsc.subcore_barrier()

    def _acc(c0, carry):
        pltpu.sync_copy(idx_hbm.at[wid, c0], idx0)
        pltpu.sync_copy(ones_v, acc.at[idx0.at[1]], add=True)
        return carry

    lax.fori_loop(0, CPT, _acc, 0)

    plsc.subcore_barrier()

    def _wb(b, carry):
        rr = t0 + b * _BROWS
        pltpu.sync_copy(acc.at[pl.ds(rr, _BROWS)], bounce)
        pltpu.sync_copy(bounce, out_hbm.at[c, pl.ds(rr, _BROWS)])
        return carry

    lax.fori_loop(0, RPT // _BROWS, _wb, 0)


@functools.partial(
    pl.kernel,
    out_type=jax.ShapeDtypeStruct((NC, NPAD, D), jnp.float32),
    mesh=_MESH,
    scratch_types=[
        pltpu.VMEM((2, CHUNK), jnp.int32),
        pltpu.VMEM((2, CHUNK), jnp.int32),
        pltpu.VMEM((CHUNK, D), jnp.float32),
        pltpu.VMEM((CHUNK, D), jnp.float32),
        pltpu.VMEM((_BROWS, D), jnp.float32),
        pltpu.VMEM_SHARED((NPAD, D), jnp.float32),
        pltpu.SemaphoreType.DMA,
        pltpu.SemaphoreType.DMA,
    ],
)
def _gs_kernel(src_hbm, idx_hbm, out_hbm,
               idx_a, idx_b, rows_a, rows_b, bounce, acc, sem_a, sem_b):
    c = lax.axis_index("c")
    s = lax.axis_index("s")
    wid = c * NS + s

    idxs = (idx_a, idx_b)
    rows = (rows_a, rows_b)
    sems = (sem_a, sem_b)

    zero16 = jnp.zeros((16,), jnp.float32)

    def _zero(i, carry):
        bounce[i // 8, pl.ds((i % 8) * 16, 16)] = zero16
        return carry

    lax.fori_loop(0, _BROWS * 8, _zero, 0)

    t0 = s * RPT

    def _zacc(b, carry):
        pltpu.sync_copy(bounce, acc.at[pl.ds(t0 + b * _BROWS, _BROWS)])
        return carry

    lax.fori_loop(0, RPT // _BROWS, _zacc, 0)
    plsc.subcore_barrier()

    for b in range(NBUF):
        pltpu.sync_copy(idx_hbm.at[wid, b], idxs[b])
        pltpu.async_copy(src_hbm.at[idxs[b].at[0]], rows[b], sems[b])

    def _body(it, carry):
        g = it * NBUF
        for b in range(NBUF):
            pltpu.make_async_copy(
                src_hbm.at[idxs[b].at[0]], rows[b], sems[b]).wait()
            pltpu.sync_copy(rows[b], acc.at[idxs[b].at[1]], add=True)
            pltpu.sync_copy(idx_hbm.at[wid, g + b + NBUF], idxs[b])
            pltpu.async_copy(src_hbm.at[idxs[b].at[0]], rows[b], sems[b])
        return carry

    lax.fori_loop(0, CPT // NBUF, _body, 0)

    for b in range(NBUF):
        pltpu.make_async_copy(
            src_hbm.at[idxs[b].at[0]], rows[b], sems[b]).wait()

    plsc.subcore_barrier()

    def _wb(b, carry):
        rr = t0 + b * _BROWS
        pltpu.sync_copy(acc.at[pl.ds(rr, _BROWS)], bounce)
        pltpu.sync_copy(bounce, out_hbm.at[c, pl.ds(rr, _BROWS)])
        return carry

    lax.fori_loop(0, RPT // _BROWS, _wb, 0)


_BLK = 1024
_GRID = NPAD // _BLK


def _feat_spec():
    return pl.BlockSpec((_BLK, D), lambda i: (i, 0))


def _col_spec():
    return pl.BlockSpec((_BLK, 1), lambda i: (i, 0))


def _fixed_spec(shape):
    return pl.BlockSpec(shape, lambda i: tuple(0 for _ in shape))


def _mm_body(x_ref, w_ref, o_ref):
    o_ref[...] = jnp.dot(x_ref[...], w_ref[...],
                         preferred_element_type=jnp.float32)


def _matmul(x, w):
    return pl.pallas_call(
        _mm_body,
        grid=(_GRID,),
        in_specs=[_feat_spec(), _fixed_spec((D, D))],
        out_specs=_feat_spec(),
        out_shape=jax.ShapeDtypeStruct((NPAD, D), jnp.float32),
    )(x, w)


def _combine_hedge_body(h0, h1, bd0, bd1, o):
    deg = bd0[...] + bd1[...]
    inv = jnp.where(deg > 0, 1.0 / deg, 0.0)
    o[...] = (h0[...] + h1[...]) * inv


def _combine_hedge(h0, h1, bd0, bd1):
    return pl.pallas_call(
        _combine_hedge_body,
        grid=(_GRID,),
        in_specs=[_feat_spec(), _feat_spec(), _col_spec(), _col_spec()],
        out_specs=_feat_spec(),
        out_shape=jax.ShapeDtypeStruct((NPAD, D), jnp.float32),
    )(h0, h1, bd0, bd1)


def _mid_body(q0, q1, dd0, dd1, b1r, w2, a, o):
    deg = dd0[...] + dd1[...]
    inv = jnp.where(deg > 0, 1.0 / deg, 0.0)
    t = (q0[...] + q1[...]) * inv + b1r[...]
    av = a[0, 0]
    t = jnp.where(t >= 0, t, av * t)
    o[...] = jnp.dot(t, w2[...], preferred_element_type=jnp.float32)


def _mid(q0, q1, dd0, dd1, b1r, w2, a):
    return pl.pallas_call(
        _mid_body,
        grid=(_GRID,),
        in_specs=[_feat_spec(), _feat_spec(), _col_spec(), _col_spec(),
                  _fixed_spec((1, D)), _fixed_spec((D, D)),
                  _fixed_spec((1, 1))],
        out_specs=_feat_spec(),
        out_shape=jax.ShapeDtypeStruct((NPAD, D), jnp.float32),
    )(q0, q1, dd0, dd1, b1r, w2, a)


def _final_body(q0, q1, dd0, dd1, b2r, xr, a, o):
    deg = dd0[...] + dd1[...]
    inv = jnp.where(deg > 0, 1.0 / deg, 0.0)
    t = (q0[...] + q1[...]) * inv + b2r[...] + xr[...]
    av = a[0, 0]
    o[...] = jnp.where(t >= 0, t, av * t)


def _final(q0, q1, dd0, dd1, b2r, xr, a):
    return pl.pallas_call(
        _final_body,
        grid=(_GRID,),
        in_specs=[_feat_spec(), _feat_spec(), _col_spec(), _col_spec(),
                  _fixed_spec((1, D)), _feat_spec(), _fixed_spec((1, 1))],
        out_specs=_feat_spec(),
        out_shape=jax.ShapeDtypeStruct((NPAD, D), jnp.float32),
    )(q0, q1, dd0, dd1, b2r, xr, a)


def kernel(x, edge_index, W1, b1, W2, b2, prelu_a):
    nidx = edge_index[0]
    hidx = edge_index[1]
    pad = jnp.full((EPAD - E,), N, dtype=jnp.int32)
    nidx_r = jnp.concatenate([nidx, pad]).reshape(TILES, CPT, CHUNK)
    hidx_r = jnp.concatenate([hidx, pad]).reshape(TILES, CPT, CHUNK)
    n2h = jnp.stack([nidx_r, hidx_r], axis=2)
    h2n = jnp.stack([hidx_r, nidx_r], axis=2)
    n2h = jnp.pad(n2h, ((0, 0), (0, NBUF), (0, 0), (0, 0)), constant_values=N)
    h2n = jnp.pad(h2n, ((0, 0), (0, NBUF), (0, 0), (0, 0)), constant_values=N)
    x_pad = jnp.pad(x, ((0, NPAD - N), (0, 0)))

    ndeg_p = _deg_kernel(h2n)
    hdeg_p = _deg_kernel(n2h)
    dd0 = ndeg_p[0, :, 0:1]
    dd1 = ndeg_p[1, :, 0:1]
    bd0 = hdeg_p[0, :, 0:1]
    bd1 = hdeg_p[1, :, 0:1]

    b1r = b1.reshape(1, D)
    b2r = b2.reshape(1, D)
    a2 = jnp.asarray(prelu_a, jnp.float32).reshape(1, 1)

    xt1 = _matmul(x_pad, W1)
    hp = _gs_kernel(xt1, n2h)
    hf1 = _combine_hedge(hp[0], hp[1], bd0, bd1)
    qp = _gs_kernel(hf1, h2n)
    xt2 = _mid(qp[0], qp[1], dd0, dd1, b1r, W2, a2)
    hp2 = _gs_kernel(xt2, n2h)
    hf2 = _combine_hedge(hp2[0], hp2[1], bd0, bd1)
    qp2 = _gs_kernel(hf2, h2n)
    out = _final(qp2[0], qp2[1], dd0, dd1, b2r, x_pad, a2)
    return out[:N]


# speedup vs baseline: 1.3125x; 1.3125x over previous
import functools

import jax
import jax.numpy as jnp
from jax import lax
from jax.experimental import pallas as pl
from jax.experimental.pallas import tpu as pltpu
from jax.experimental.pallas import tpu_sc as plsc

N = 10000
E = 320000
D = 128
NC = 2
NS = 16
TILES = NC * NS
NPAD = 10240
CHUNK = 128
CPT = 80
EPAD = TILES * CPT * CHUNK
RPT = NPAD // NS
NBUF = 2
CPAD = CPT + NBUF

_MESH = plsc.VectorSubcoreMesh(
    core_axis_name="c", subcore_axis_name="s", num_cores=NC, num_subcores=NS
)


_BROWS = 64


@functools.partial(
    pl.kernel,
    out_type=jax.ShapeDtypeStruct((NC, NPAD, D), jnp.float32),
    mesh=_MESH,
    scratch_types=[
        pltpu.VMEM((2, CHUNK), jnp.int32),
        pltpu.VMEM((CHUNK, D), jnp.float32),
        pltpu.VMEM((_BROWS, D), jnp.float32),
        pltpu.VMEM_SHARED((NPAD, D), jnp.float32),
    ],
)
def _deg_kernel(idx_hbm, out_hbm, idx0, ones_v, bounce, acc):
    c = lax.axis_index("c")
    s = lax.axis_index("s")
    wid = c * NS + s

    ones16 = jnp.full((16,), 1.0, jnp.float32)
    zero16 = jnp.zeros((16,), jnp.float32)

    def _fill(i, carry):
        ones_v[i // 8, pl.ds((i % 8) * 16, 16)] = ones16
        return carry

    lax.fori_loop(0, CHUNK * 8, _fill, 0)

    def _zero(i, carry):
        bounce[i // 8, pl.ds((i % 8) * 16, 16)] = zero16
        return carry

    lax.fori_loop(0, _BROWS * 8, _zero, 0)

    t0 = s * RPT

    def _zacc(b, carry):
        pltpu.sync_copy(bounce, acc.at[pl.ds(t0 + b * _BROWS, _BROWS)])
        return carry

    lax.fori_loop(0, RPT // _BROWS, _zacc, 0)
    plsc.subcore_barrier()

    def _acc(c0, carry):
        pltpu.sync_copy(idx_hbm.at[wid, c0], idx0)
        pltpu.sync_copy(ones_v, acc.at[idx0.at[1]], add=True)
        return carry

    lax.fori_loop(0, CPT, _acc, 0)

    plsc.subcore_barrier()

    def _wb(b, carry):
        rr = t0 + b * _BROWS
        pltpu.sync_copy(acc.at[pl.ds(rr, _BROWS)], bounce)
        pltpu.sync_copy(bounce, out_hbm.at[c, pl.ds(rr, _BROWS)])
        return carry

    lax.fori_loop(0, RPT // _BROWS, _wb, 0)


@functools.partial(
    pl.kernel,
    out_type=jax.ShapeDtypeStruct((NC, NPAD, D), jnp.float32),
    mesh=_MESH,
    scratch_types=[
        pltpu.VMEM((2, CHUNK), jnp.int32),
        pltpu.VMEM((CHUNK, D), jnp.float32),
        pltpu.VMEM((_BROWS, D), jnp.float32),
        pltpu.VMEM_SHARED((NPAD, D), jnp.float32),
        pltpu.SemaphoreType.DMA,
    ],
)
def _gs_kernel(src_hbm, idx_hbm, out_hbm,
               idx_a, rows_a, bounce, acc, sem_a):
    c = lax.axis_index("c")
    s = lax.axis_index("s")
    wid = c * NS + s

    zero16 = jnp.zeros((16,), jnp.float32)

    def _zero(i, carry):
        bounce[i // 8, pl.ds((i % 8) * 16, 16)] = zero16
        return carry

    lax.fori_loop(0, _BROWS * 8, _zero, 0)

    t0 = s * RPT

    def _zacc(b, carry):
        pltpu.sync_copy(bounce, acc.at[pl.ds(t0 + b * _BROWS, _BROWS)])
        return carry

    lax.fori_loop(0, RPT // _BROWS, _zacc, 0)
    plsc.subcore_barrier()

    def _body(c0, carry):
        pltpu.sync_copy(idx_hbm.at[wid, c0], idx_a)
        pltpu.async_copy(src_hbm.at[idx_a.at[0]], rows_a, sem_a).wait()
        pltpu.sync_copy(rows_a, acc.at[idx_a.at[1]], add=True)
        return carry

    lax.fori_loop(0, CPT, _body, 0)

    plsc.subcore_barrier()

    def _wb(b, carry):
        rr = t0 + b * _BROWS
        pltpu.sync_copy(acc.at[pl.ds(rr, _BROWS)], bounce)
        pltpu.sync_copy(bounce, out_hbm.at[c, pl.ds(rr, _BROWS)])
        return carry

    lax.fori_loop(0, RPT // _BROWS, _wb, 0)


_BLK = 1024
_GRID = NPAD // _BLK


def _feat_spec():
    return pl.BlockSpec((_BLK, D), lambda i: (i, 0))


def _col_spec():
    return pl.BlockSpec((_BLK, 1), lambda i: (i, 0))


def _fixed_spec(shape):
    return pl.BlockSpec(shape, lambda i: tuple(0 for _ in shape))


def _mm_body(x_ref, w_ref, o_ref):
    o_ref[...] = jnp.dot(x_ref[...], w_ref[...],
                         preferred_element_type=jnp.float32)


def _matmul(x, w):
    return pl.pallas_call(
        _mm_body,
        grid=(_GRID,),
        in_specs=[_feat_spec(), _fixed_spec((D, D))],
        out_specs=_feat_spec(),
        out_shape=jax.ShapeDtypeStruct((NPAD, D), jnp.float32),
    )(x, w)


def _combine_hedge_body(h0, h1, bd0, bd1, o):
    deg = bd0[...] + bd1[...]
    inv = jnp.where(deg > 0, 1.0 / deg, 0.0)
    o[...] = (h0[...] + h1[...]) * inv


def _combine_hedge(h0, h1, bd0, bd1):
    return pl.pallas_call(
        _combine_hedge_body,
        grid=(_GRID,),
        in_specs=[_feat_spec(), _feat_spec(), _col_spec(), _col_spec()],
        out_specs=_feat_spec(),
        out_shape=jax.ShapeDtypeStruct((NPAD, D), jnp.float32),
    )(h0, h1, bd0, bd1)


def _mid_body(q0, q1, dd0, dd1, b1r, w2, a, o):
    deg = dd0[...] + dd1[...]
    inv = jnp.where(deg > 0, 1.0 / deg, 0.0)
    t = (q0[...] + q1[...]) * inv + b1r[...]
    av = a[0, 0]
    t = jnp.where(t >= 0, t, av * t)
    o[...] = jnp.dot(t, w2[...], preferred_element_type=jnp.float32)


def _mid(q0, q1, dd0, dd1, b1r, w2, a):
    return pl.pallas_call(
        _mid_body,
        grid=(_GRID,),
        in_specs=[_feat_spec(), _feat_spec(), _col_spec(), _col_spec(),
                  _fixed_spec((1, D)), _fixed_spec((D, D)),
                  _fixed_spec((1, 1))],
        out_specs=_feat_spec(),
        out_shape=jax.ShapeDtypeStruct((NPAD, D), jnp.float32),
    )(q0, q1, dd0, dd1, b1r, w2, a)


def _final_body(q0, q1, dd0, dd1, b2r, xr, a, o):
    deg = dd0[...] + dd1[...]
    inv = jnp.where(deg > 0, 1.0 / deg, 0.0)
    t = (q0[...] + q1[...]) * inv + b2r[...] + xr[...]
    av = a[0, 0]
    o[...] = jnp.where(t >= 0, t, av * t)


def _final(q0, q1, dd0, dd1, b2r, xr, a):
    return pl.pallas_call(
        _final_body,
        grid=(_GRID,),
        in_specs=[_feat_spec(), _feat_spec(), _col_spec(), _col_spec(),
                  _fixed_spec((1, D)), _feat_spec(), _fixed_spec((1, 1))],
        out_specs=_feat_spec(),
        out_shape=jax.ShapeDtypeStruct((NPAD, D), jnp.float32),
    )(q0, q1, dd0, dd1, b2r, xr, a)


def kernel(x, edge_index, W1, b1, W2, b2, prelu_a):
    nidx = edge_index[0]
    hidx = edge_index[1]
    pad = jnp.full((EPAD - E,), N, dtype=jnp.int32)
    nidx_r = jnp.concatenate([nidx, pad]).reshape(TILES, CPT, CHUNK)
    hidx_r = jnp.concatenate([hidx, pad]).reshape(TILES, CPT, CHUNK)
    n2h = jnp.stack([nidx_r, hidx_r], axis=2)
    h2n = jnp.stack([hidx_r, nidx_r], axis=2)
    n2h = jnp.pad(n2h, ((0, 0), (0, NBUF), (0, 0), (0, 0)), constant_values=N)
    h2n = jnp.pad(h2n, ((0, 0), (0, NBUF), (0, 0), (0, 0)), constant_values=N)
    x_pad = jnp.pad(x, ((0, NPAD - N), (0, 0)))

    ndeg_p = _deg_kernel(h2n)
    hdeg_p = _deg_kernel(n2h)
    dd0 = ndeg_p[0, :, 0:1]
    dd1 = ndeg_p[1, :, 0:1]
    bd0 = hdeg_p[0, :, 0:1]
    bd1 = hdeg_p[1, :, 0:1]

    b1r = b1.reshape(1, D)
    b2r = b2.reshape(1, D)
    a2 = jnp.asarray(prelu_a, jnp.float32).reshape(1, 1)

    xt1 = _matmul(x_pad, W1)
    hp = _gs_kernel(xt1, n2h)
    hf1 = _combine_hedge(hp[0], hp[1], bd0, bd1)
    qp = _gs_kernel(hf1, h2n)
    xt2 = _mid(qp[0], qp[1], dd0, dd1, b1r, W2, a2)
    hp2 = _gs_kernel(xt2, n2h)
    hf2 = _combine_hedge(hp2[0], hp2[1], bd0, bd1)
    qp2 = _gs_kernel(hf2, h2n)
    out = _final(qp2[0], qp2[1], dd0, dd1, b2r, x_pad, a2)
    return out[:N]


# speedup vs baseline: 3.4302x; 2.6136x over previous
import functools

import jax
import jax.numpy as jnp
from jax import lax
from jax.experimental import pallas as pl
from jax.experimental.pallas import tpu as pltpu
from jax.experimental.pallas import tpu_sc as plsc

N = 10000
E = 320000
D = 128
NC = 2
NS = 16
TILES = NC * NS
NPAD = 10240
CHUNK = 128
CPT = 80
EPAD = TILES * CPT * CHUNK
RPT = NPAD // NS

_MESH = plsc.VectorSubcoreMesh(
    core_axis_name="c", subcore_axis_name="s", num_cores=NC, num_subcores=NS
)


_BROWS = 64
_DW = D


@functools.partial(
    pl.kernel,
    out_type=jax.ShapeDtypeStruct((NC, NPAD, _DW), jnp.float32),
    mesh=_MESH,
    scratch_types=[
        pltpu.VMEM((2, CHUNK), jnp.int32),
        pltpu.VMEM((CHUNK, _DW), jnp.float32),
        pltpu.VMEM((_BROWS, _DW), jnp.float32),
        pltpu.VMEM_SHARED((NPAD, _DW), jnp.float32),
    ],
)
def _deg_kernel(idx_hbm, out_hbm, idx0, ones_v, bounce, acc):
    c = lax.axis_index("c")
    s = lax.axis_index("s")
    wid = c * NS + s

    ones16 = jnp.full((16,), 1.0, jnp.float32)
    zero16 = jnp.zeros((16,), jnp.float32)

    def _fill(i, carry):
        ones_v[i // 8, pl.ds((i % 8) * 16, 16)] = ones16
        return carry

    lax.fori_loop(0, CHUNK * 8, _fill, 0)

    def _zero(i, carry):
        bounce[i // 8, pl.ds((i % 8) * 16, 16)] = zero16
        return carry

    lax.fori_loop(0, _BROWS * 8, _zero, 0)

    t0 = s * RPT

    def _zacc(b, carry):
        pltpu.sync_copy(bounce, acc.at[pl.ds(t0 + b * _BROWS, _BROWS)])
        return carry

    lax.fori_loop(0, RPT // _BROWS, _zacc, 0)
    plsc.subcore_barrier()

    def _acc(c0, carry):
        pltpu.sync_copy(idx_hbm.at[wid, c0], idx0)
        pltpu.sync_copy(ones_v, acc.at[idx0.at[1]], add=True)
        return carry

    lax.fori_loop(0, CPT, _acc, 0)

    plsc.subcore_barrier()

    def _wb(b, carry):
        rr = t0 + b * _BROWS
        pltpu.sync_copy(acc.at[pl.ds(rr, _BROWS)], bounce)
        pltpu.sync_copy(bounce, out_hbm.at[c, pl.ds(rr, _BROWS)])
        return carry

    lax.fori_loop(0, RPT // _BROWS, _wb, 0)


@functools.partial(
    pl.kernel,
    out_type=jax.ShapeDtypeStruct((NC, NPAD, D), jnp.float32),
    mesh=_MESH,
    scratch_types=[
        pltpu.VMEM((CPT, 2, CHUNK), jnp.int32),
        pltpu.VMEM((CHUNK, D), jnp.float32),
        pltpu.VMEM((_BROWS, D), jnp.float32),
        pltpu.VMEM_SHARED((NPAD, D), jnp.float32),
        pltpu.SemaphoreType.DMA,
    ],
)
def _gs_kernel(src_hbm, idx_hbm, out_hbm,
               idx_t, rows_a, bounce, acc, sem_a):
    c = lax.axis_index("c")
    s = lax.axis_index("s")
    wid = c * NS + s

    pltpu.sync_copy(idx_hbm.at[wid], idx_t)

    zero16 = jnp.zeros((16,), jnp.float32)

    def _zero(i, carry):
        bounce[i // 8, pl.ds((i % 8) * 16, 16)] = zero16
        return carry

    lax.fori_loop(0, _BROWS * 8, _zero, 0)

    t0 = s * RPT

    def _zacc(b, carry):
        pltpu.sync_copy(bounce, acc.at[pl.ds(t0 + b * _BROWS, _BROWS)])
        return carry

    lax.fori_loop(0, RPT // _BROWS, _zacc, 0)
    plsc.subcore_barrier()

    def _body(c0, carry):
        idx_c = idx_t.at[c0]
        pltpu.async_copy(src_hbm.at[idx_c.at[0]], rows_a, sem_a).wait()
        pltpu.sync_copy(rows_a, acc.at[idx_c.at[1]], add=True)
        return carry

    lax.fori_loop(0, CPT, _body, 0)

    plsc.subcore_barrier()

    def _wb(b, carry):
        rr = t0 + b * _BROWS
        pltpu.sync_copy(acc.at[pl.ds(rr, _BROWS)], bounce)
        pltpu.sync_copy(bounce, out_hbm.at[c, pl.ds(rr, _BROWS)])
        return carry

    lax.fori_loop(0, RPT // _BROWS, _wb, 0)


_BLK = 1024
_GRID = NPAD // _BLK


def _feat_spec():
    return pl.BlockSpec((_BLK, D), lambda i: (i, 0))


def _col_spec():
    return pl.BlockSpec((_BLK, 1), lambda i: (i, 0))


def _fixed_spec(shape):
    return pl.BlockSpec(shape, lambda i: tuple(0 for _ in shape))


def _mm_body(x_ref, w_ref, o_ref):
    o_ref[...] = jnp.dot(x_ref[...], w_ref[...],
                         preferred_element_type=jnp.float32)


def _matmul(x, w):
    return pl.pallas_call(
        _mm_body,
        grid=(_GRID,),
        in_specs=[_feat_spec(), _fixed_spec((D, D))],
        out_specs=_feat_spec(),
        out_shape=jax.ShapeDtypeStruct((NPAD, D), jnp.float32),
    )(x, w)


def _combine_hedge_body(h0, h1, bd0, bd1, o):
    deg = bd0[...] + bd1[...]
    inv = jnp.where(deg > 0, 1.0 / deg, 0.0)
    o[...] = (h0[...] + h1[...]) * inv


def _combine_hedge(h0, h1, bd0, bd1):
    return pl.pallas_call(
        _combine_hedge_body,
        grid=(_GRID,),
        in_specs=[_feat_spec(), _feat_spec(), _col_spec(), _col_spec()],
        out_specs=_feat_spec(),
        out_shape=jax.ShapeDtypeStruct((NPAD, D), jnp.float32),
    )(h0, h1, bd0, bd1)


def _mid_body(q0, q1, dd0, dd1, b1r, w2, a, o):
    deg = dd0[...] + dd1[...]
    inv = jnp.where(deg > 0, 1.0 / deg, 0.0)
    t = (q0[...] + q1[...]) * inv + b1r[...]
    av = a[0, 0]
    t = jnp.where(t >= 0, t, av * t)
    o[...] = jnp.dot(t, w2[...], preferred_element_type=jnp.float32)


def _mid(q0, q1, dd0, dd1, b1r, w2, a):
    return pl.pallas_call(
        _mid_body,
        grid=(_GRID,),
        in_specs=[_feat_spec(), _feat_spec(), _col_spec(), _col_spec(),
                  _fixed_spec((1, D)), _fixed_spec((D, D)),
                  _fixed_spec((1, 1))],
        out_specs=_feat_spec(),
        out_shape=jax.ShapeDtypeStruct((NPAD, D), jnp.float32),
    )(q0, q1, dd0, dd1, b1r, w2, a)


def _final_body(q0, q1, dd0, dd1, b2r, xr, a, o):
    deg = dd0[...] + dd1[...]
    inv = jnp.where(deg > 0, 1.0 / deg, 0.0)
    t = (q0[...] + q1[...]) * inv + b2r[...] + xr[...]
    av = a[0, 0]
    o[...] = jnp.where(t >= 0, t, av * t)


def _final(q0, q1, dd0, dd1, b2r, xr, a):
    return pl.pallas_call(
        _final_body,
        grid=(_GRID,),
        in_specs=[_feat_spec(), _feat_spec(), _col_spec(), _col_spec(),
                  _fixed_spec((1, D)), _feat_spec(), _fixed_spec((1, 1))],
        out_specs=_feat_spec(),
        out_shape=jax.ShapeDtypeStruct((NPAD, D), jnp.float32),
    )(q0, q1, dd0, dd1, b2r, xr, a)


def kernel(x, edge_index, W1, b1, W2, b2, prelu_a):
    nidx = edge_index[0]
    hidx = edge_index[1]
    pad = N + (jnp.arange(EPAD - E, dtype=jnp.int32) % (NPAD - N))
    nidx_r = jnp.concatenate([nidx, pad]).reshape(TILES, CPT, CHUNK)
    hidx_r = jnp.concatenate([hidx, pad]).reshape(TILES, CPT, CHUNK)
    n2h = jnp.stack([nidx_r, hidx_r], axis=2)
    h2n = jnp.stack([hidx_r, nidx_r], axis=2)
    x_pad = jnp.pad(x, ((0, NPAD - N), (0, 0)))

    ndeg_p = _deg_kernel(h2n)
    hdeg_p = _deg_kernel(n2h)
    dd0 = ndeg_p[0, :, 0:1]
    dd1 = ndeg_p[1, :, 0:1]
    bd0 = hdeg_p[0, :, 0:1]
    bd1 = hdeg_p[1, :, 0:1]

    b1r = b1.reshape(1, D)
    b2r = b2.reshape(1, D)
    a2 = jnp.asarray(prelu_a, jnp.float32).reshape(1, 1)

    xt1 = _matmul(x_pad, W1)
    hp = _gs_kernel(xt1, n2h)
    hf1 = _combine_hedge(hp[0], hp[1], bd0, bd1)
    qp = _gs_kernel(hf1, h2n)
    xt2 = _mid(qp[0], qp[1], dd0, dd1, b1r, W2, a2)
    hp2 = _gs_kernel(xt2, n2h)
    hf2 = _combine_hedge(hp2[0], hp2[1], bd0, bd1)
    qp2 = _gs_kernel(hf2, h2n)
    out = _final(qp2[0], qp2[1], dd0, dd1, b2r, x_pad, a2)
    return out[:N]


# speedup vs baseline: 4.5985x; 1.3406x over previous
import functools

import jax
import jax.numpy as jnp
from jax import lax
from jax.experimental import pallas as pl
from jax.experimental.pallas import tpu as pltpu
from jax.experimental.pallas import tpu_sc as plsc

N = 10000
E = 320000
D = 128
NC = 2
NS = 16
TILES = NC * NS
NPAD = 10240
CHUNK = 128
CPT = 80
EPAD = TILES * CPT * CHUNK
RPT = NPAD // NS

_MESH = plsc.VectorSubcoreMesh(
    core_axis_name="c", subcore_axis_name="s", num_cores=NC, num_subcores=NS
)


_BROWS = 64
_DW = D


@functools.partial(
    pl.kernel,
    out_type=jax.ShapeDtypeStruct((NC, NPAD, _DW), jnp.float32),
    mesh=_MESH,
    scratch_types=[
        pltpu.VMEM((2, CHUNK), jnp.int32),
        pltpu.VMEM((CHUNK, _DW), jnp.float32),
        pltpu.VMEM((_BROWS, _DW), jnp.float32),
        pltpu.VMEM_SHARED((NPAD, _DW), jnp.float32),
    ],
)
def _deg_kernel(idx_hbm, out_hbm, idx0, ones_v, bounce, acc):
    c = lax.axis_index("c")
    s = lax.axis_index("s")
    wid = c * NS + s

    ones16 = jnp.full((16,), 1.0, jnp.float32)
    zero16 = jnp.zeros((16,), jnp.float32)

    def _fill(i, carry):
        ones_v[i // 8, pl.ds((i % 8) * 16, 16)] = ones16
        return carry

    lax.fori_loop(0, CHUNK * 8, _fill, 0)

    def _zero(i, carry):
        bounce[i // 8, pl.ds((i % 8) * 16, 16)] = zero16
        return carry

    lax.fori_loop(0, _BROWS * 8, _zero, 0)

    t0 = s * RPT

    def _zacc(b, carry):
        pltpu.sync_copy(bounce, acc.at[pl.ds(t0 + b * _BROWS, _BROWS)])
        return carry

    lax.fori_loop(0, RPT // _BROWS, _zacc, 0)
    plsc.subcore_barrier()

    def _acc(c0, carry):
        pltpu.sync_copy(idx_hbm.at[wid, c0], idx0)
        pltpu.sync_copy(ones_v, acc.at[idx0.at[1]], add=True)
        return carry

    lax.fori_loop(0, CPT, _acc, 0)

    plsc.subcore_barrier()

    def _wb(b, carry):
        rr = t0 + b * _BROWS
        pltpu.sync_copy(acc.at[pl.ds(rr, _BROWS)], bounce)
        pltpu.sync_copy(bounce, out_hbm.at[c, pl.ds(rr, _BROWS)])
        return carry

    lax.fori_loop(0, RPT // _BROWS, _wb, 0)


_HALF = CPT // 2


@functools.partial(
    pl.kernel,
    out_type=jax.ShapeDtypeStruct((NC, NPAD, D), jnp.float32),
    mesh=_MESH,
    scratch_types=[
        pltpu.VMEM((_HALF, 2, CHUNK), jnp.int32),
        pltpu.VMEM((CHUNK, D), jnp.float32),
        pltpu.VMEM((CHUNK, D), jnp.float32),
        pltpu.VMEM_SHARED((NPAD, D), jnp.float32),
        pltpu.SemaphoreType.DMA,
        pltpu.SemaphoreType.DMA,
    ],
)
def _gs_kernel(src_hbm, idx_hbm, out_hbm,
               idx_t, rows_a, rows_b, acc, sem_a, sem_b):
    c = lax.axis_index("c")
    s = lax.axis_index("s")
    wid = c * NS + s

    zero16 = jnp.zeros((16,), jnp.float32)

    def _zero(i, carry):
        rows_a[i // 8, pl.ds((i % 8) * 16, 16)] = zero16
        return carry

    lax.fori_loop(0, CHUNK * 8, _zero, 0)

    t0 = s * RPT

    def _zacc(b, carry):
        pltpu.sync_copy(rows_a, acc.at[pl.ds(t0 + b * CHUNK, CHUNK)])
        return carry

    lax.fori_loop(0, RPT // CHUNK, _zacc, 0)
    plsc.subcore_barrier()

    for h in range(2):
        pltpu.sync_copy(idx_hbm.at[wid, pl.ds(h * _HALF, _HALF)], idx_t)

        pltpu.async_copy(src_hbm.at[idx_t.at[0].at[0]], rows_a, sem_a)

        def _body(g, carry):
            c0 = 2 * g
            i0 = idx_t.at[c0]
            i1 = idx_t.at[c0 + 1]
            i2 = idx_t.at[c0 + 2]
            pltpu.async_copy(src_hbm.at[i1.at[0]], rows_b, sem_b)
            pltpu.make_async_copy(src_hbm.at[i0.at[0]], rows_a, sem_a).wait()
            pltpu.sync_copy(rows_a, acc.at[i0.at[1]], add=True)
            pltpu.async_copy(src_hbm.at[i2.at[0]], rows_a, sem_a)
            pltpu.make_async_copy(src_hbm.at[i1.at[0]], rows_b, sem_b).wait()
            pltpu.sync_copy(rows_b, acc.at[i1.at[1]], add=True)
            return carry

        lax.fori_loop(0, _HALF // 2 - 1, _body, 0)

        ia = idx_t.at[_HALF - 2]
        ib = idx_t.at[_HALF - 1]
        pltpu.async_copy(src_hbm.at[ib.at[0]], rows_b, sem_b)
        pltpu.make_async_copy(src_hbm.at[ia.at[0]], rows_a, sem_a).wait()
        pltpu.sync_copy(rows_a, acc.at[ia.at[1]], add=True)
        pltpu.make_async_copy(src_hbm.at[ib.at[0]], rows_b, sem_b).wait()
        pltpu.sync_copy(rows_b, acc.at[ib.at[1]], add=True)

    plsc.subcore_barrier()

    def _wb(b, carry):
        rr = t0 + b * CHUNK
        pltpu.sync_copy(acc.at[pl.ds(rr, CHUNK)], rows_a)
        pltpu.sync_copy(rows_a, out_hbm.at[c, pl.ds(rr, CHUNK)])
        return carry

    lax.fori_loop(0, RPT // CHUNK, _wb, 0)


_BLK = 1024
_GRID = NPAD // _BLK


def _feat_spec():
    return pl.BlockSpec((_BLK, D), lambda i: (i, 0))


def _col_spec():
    return pl.BlockSpec((_BLK, 1), lambda i: (i, 0))


def _fixed_spec(shape):
    return pl.BlockSpec(shape, lambda i: tuple(0 for _ in shape))


def _mm_body(x_ref, w_ref, o_ref):
    o_ref[...] = jnp.dot(x_ref[...], w_ref[...],
                         preferred_element_type=jnp.float32)


def _matmul(x, w):
    return pl.pallas_call(
        _mm_body,
        grid=(_GRID,),
        in_specs=[_feat_spec(), _fixed_spec((D, D))],
        out_specs=_feat_spec(),
        out_shape=jax.ShapeDtypeStruct((NPAD, D), jnp.float32),
    )(x, w)


def _combine_hedge_body(h0, h1, bd0, bd1, o):
    deg = bd0[...] + bd1[...]
    inv = jnp.where(deg > 0, 1.0 / deg, 0.0)
    o[...] = (h0[...] + h1[...]) * inv


def _combine_hedge(h0, h1, bd0, bd1):
    return pl.pallas_call(
        _combine_hedge_body,
        grid=(_GRID,),
        in_specs=[_feat_spec(), _feat_spec(), _col_spec(), _col_spec()],
        out_specs=_feat_spec(),
        out_shape=jax.ShapeDtypeStruct((NPAD, D), jnp.float32),
    )(h0, h1, bd0, bd1)


def _mid_body(q0, q1, dd0, dd1, b1r, w2, a, o):
    deg = dd0[...] + dd1[...]
    inv = jnp.where(deg > 0, 1.0 / deg, 0.0)
    t = (q0[...] + q1[...]) * inv + b1r[...]
    av = a[0, 0]
    t = jnp.where(t >= 0, t, av * t)
    o[...] = jnp.dot(t, w2[...], preferred_element_type=jnp.float32)


def _mid(q0, q1, dd0, dd1, b1r, w2, a):
    return pl.pallas_call(
        _mid_body,
        grid=(_GRID,),
        in_specs=[_feat_spec(), _feat_spec(), _col_spec(), _col_spec(),
                  _fixed_spec((1, D)), _fixed_spec((D, D)),
                  _fixed_spec((1, 1))],
        out_specs=_feat_spec(),
        out_shape=jax.ShapeDtypeStruct((NPAD, D), jnp.float32),
    )(q0, q1, dd0, dd1, b1r, w2, a)


def _final_body(q0, q1, dd0, dd1, b2r, xr, a, o):
    deg = dd0[...] + dd1[...]
    inv = jnp.where(deg > 0, 1.0 / deg, 0.0)
    t = (q0[...] + q1[...]) * inv + b2r[...] + xr[...]
    av = a[0, 0]
    o[...] = jnp.where(t >= 0, t, av * t)


def _final(q0, q1, dd0, dd1, b2r, xr, a):
    return pl.pallas_call(
        _final_body,
        grid=(_GRID,),
        in_specs=[_feat_spec(), _feat_spec(), _col_spec(), _col_spec(),
                  _fixed_spec((1, D)), _feat_spec(), _fixed_spec((1, 1))],
        out_specs=_feat_spec(),
        out_shape=jax.ShapeDtypeStruct((NPAD, D), jnp.float32),
    )(q0, q1, dd0, dd1, b2r, xr, a)


def kernel(x, edge_index, W1, b1, W2, b2, prelu_a):
    nidx = edge_index[0]
    hidx = edge_index[1]
    pad = N + (jnp.arange(EPAD - E, dtype=jnp.int32) % (NPAD - N))
    nidx_r = jnp.concatenate([nidx, pad]).reshape(TILES, CPT, CHUNK)
    hidx_r = jnp.concatenate([hidx, pad]).reshape(TILES, CPT, CHUNK)
    n2h = jnp.stack([nidx_r, hidx_r], axis=2)
    h2n = jnp.stack([hidx_r, nidx_r], axis=2)
    x_pad = jnp.pad(x, ((0, NPAD - N), (0, 0)))

    ndeg_p = _deg_kernel(h2n)
    hdeg_p = _deg_kernel(n2h)
    dd0 = ndeg_p[0, :, 0:1]
    dd1 = ndeg_p[1, :, 0:1]
    bd0 = hdeg_p[0, :, 0:1]
    bd1 = hdeg_p[1, :, 0:1]

    b1r = b1.reshape(1, D)
    b2r = b2.reshape(1, D)
    a2 = jnp.asarray(prelu_a, jnp.float32).reshape(1, 1)

    xt1 = _matmul(x_pad, W1)
    hp = _gs_kernel(xt1, n2h)
    hf1 = _combine_hedge(hp[0], hp[1], bd0, bd1)
    qp = _gs_kernel(hf1, h2n)
    xt2 = _mid(qp[0], qp[1], dd0, dd1, b1r, W2, a2)
    hp2 = _gs_kernel(xt2, n2h)
    hf2 = _combine_hedge(hp2[0], hp2[1], bd0, bd1)
    qp2 = _gs_kernel(hf2, h2n)
    out = _final(qp2[0], qp2[1], dd0, dd1, b2r, x_pad, a2)
    return out[:N]


# speedup vs baseline: 5.1236x; 1.1142x over previous
import functools

import jax
import jax.numpy as jnp
from jax import lax
from jax.experimental import pallas as pl
from jax.experimental.pallas import tpu as pltpu
from jax.experimental.pallas import tpu_sc as plsc

N = 10000
E = 320000
D = 128
NC = 2
NS = 16
TILES = NC * NS
NPAD = 10240
CHUNK = 128
CPT = 80
EPAD = TILES * CPT * CHUNK
RPT = NPAD // NS

_MESH = plsc.VectorSubcoreMesh(
    core_axis_name="c", subcore_axis_name="s", num_cores=NC, num_subcores=NS
)


_BROWS = 64
_DCPT = TILES * CPT // NS


@functools.partial(
    pl.kernel,
    out_type=jax.ShapeDtypeStruct((NC, NPAD, D), jnp.float32),
    mesh=_MESH,
    scratch_types=[
        pltpu.VMEM((_DCPT, CHUNK), jnp.int32),
        pltpu.VMEM((CHUNK, D), jnp.float32),
        pltpu.VMEM((_BROWS, D), jnp.float32),
        pltpu.VMEM_SHARED((NPAD, D), jnp.float32),
    ],
)
def _deg_kernel(idx_hbm, out_hbm, idx_t, ones_v, bounce, acc):
    c = lax.axis_index("c")
    s = lax.axis_index("s")

    pltpu.sync_copy(idx_hbm.at[c, s], idx_t)

    ones16 = jnp.full((16,), 1.0, jnp.float32)
    zero16 = jnp.zeros((16,), jnp.float32)

    def _fill(i, carry):
        ones_v[i // 8, pl.ds((i % 8) * 16, 16)] = ones16
        return carry

    lax.fori_loop(0, CHUNK * 8, _fill, 0)

    def _zero(i, carry):
        bounce[i // 8, pl.ds((i % 8) * 16, 16)] = zero16
        return carry

    lax.fori_loop(0, _BROWS * 8, _zero, 0)

    t0 = s * RPT

    def _zacc(b, carry):
        pltpu.sync_copy(bounce, acc.at[pl.ds(t0 + b * _BROWS, _BROWS)])
        return carry

    lax.fori_loop(0, RPT // _BROWS, _zacc, 0)
    plsc.subcore_barrier()

    def _acc(c0, carry):
        pltpu.sync_copy(ones_v, acc.at[idx_t.at[c0]], add=True)
        return carry

    lax.fori_loop(0, _DCPT, _acc, 0)

    plsc.subcore_barrier()

    def _wb(b, carry):
        rr = t0 + b * _BROWS
        pltpu.sync_copy(acc.at[pl.ds(rr, _BROWS)], bounce)
        pltpu.sync_copy(bounce, out_hbm.at[c, pl.ds(rr, _BROWS)])
        return carry

    lax.fori_loop(0, RPT // _BROWS, _wb, 0)


_HALF = CPT // 2


@functools.partial(
    pl.kernel,
    out_type=jax.ShapeDtypeStruct((NC, NPAD, D), jnp.float32),
    mesh=_MESH,
    scratch_types=[
        pltpu.VMEM((_HALF, 2, CHUNK), jnp.int32),
        pltpu.VMEM((CHUNK, D), jnp.float32),
        pltpu.VMEM((CHUNK, D), jnp.float32),
        pltpu.VMEM_SHARED((NPAD, D), jnp.float32),
        pltpu.SemaphoreType.DMA,
        pltpu.SemaphoreType.DMA,
    ],
)
def _gs_kernel(src_hbm, idx_hbm, out_hbm,
               idx_t, rows_a, rows_b, acc, sem_a, sem_b):
    c = lax.axis_index("c")
    s = lax.axis_index("s")
    wid = c * NS + s

    zero16 = jnp.zeros((16,), jnp.float32)

    def _zero(i, carry):
        rows_a[i // 8, pl.ds((i % 8) * 16, 16)] = zero16
        return carry

    lax.fori_loop(0, CHUNK * 8, _zero, 0)

    t0 = s * RPT

    def _zacc(b, carry):
        pltpu.sync_copy(rows_a, acc.at[pl.ds(t0 + b * CHUNK, CHUNK)])
        return carry

    lax.fori_loop(0, RPT // CHUNK, _zacc, 0)
    plsc.subcore_barrier()

    for h in range(2):
        pltpu.sync_copy(idx_hbm.at[wid, pl.ds(h * _HALF, _HALF)], idx_t)

        pltpu.async_copy(src_hbm.at[idx_t.at[0].at[0]], rows_a, sem_a)

        def _body(g, carry):
            c0 = 2 * g
            i0 = idx_t.at[c0]
            i1 = idx_t.at[c0 + 1]
            i2 = idx_t.at[c0 + 2]
            pltpu.async_copy(src_hbm.at[i1.at[0]], rows_b, sem_b)
            pltpu.make_async_copy(src_hbm.at[i0.at[0]], rows_a, sem_a).wait()
            pltpu.sync_copy(rows_a, acc.at[i0.at[1]], add=True)
            pltpu.async_copy(src_hbm.at[i2.at[0]], rows_a, sem_a)
            pltpu.make_async_copy(src_hbm.at[i1.at[0]], rows_b, sem_b).wait()
            pltpu.sync_copy(rows_b, acc.at[i1.at[1]], add=True)
            return carry

        lax.fori_loop(0, _HALF // 2 - 1, _body, 0)

        ia = idx_t.at[_HALF - 2]
        ib = idx_t.at[_HALF - 1]
        pltpu.async_copy(src_hbm.at[ib.at[0]], rows_b, sem_b)
        pltpu.make_async_copy(src_hbm.at[ia.at[0]], rows_a, sem_a).wait()
        pltpu.sync_copy(rows_a, acc.at[ia.at[1]], add=True)
        pltpu.make_async_copy(src_hbm.at[ib.at[0]], rows_b, sem_b).wait()
        pltpu.sync_copy(rows_b, acc.at[ib.at[1]], add=True)

    plsc.subcore_barrier()

    def _wb(b, carry):
        rr = t0 + b * CHUNK
        pltpu.sync_copy(acc.at[pl.ds(rr, CHUNK)], rows_a)
        pltpu.sync_copy(rows_a, out_hbm.at[c, pl.ds(rr, CHUNK)])
        return carry

    lax.fori_loop(0, RPT // CHUNK, _wb, 0)


_BLK = 1024
_GRID = NPAD // _BLK


def _feat_spec():
    return pl.BlockSpec((_BLK, D), lambda i: (i, 0))


def _col_spec():
    return pl.BlockSpec((_BLK, 1), lambda i: (i, 0))


def _fixed_spec(shape):
    return pl.BlockSpec(shape, lambda i: tuple(0 for _ in shape))


def _mm_body(x_ref, w_ref, o_ref):
    o_ref[...] = jnp.dot(x_ref[...], w_ref[...],
                         preferred_element_type=jnp.float32)


def _matmul(x, w):
    return pl.pallas_call(
        _mm_body,
        grid=(_GRID,),
        in_specs=[_feat_spec(), _fixed_spec((D, D))],
        out_specs=_feat_spec(),
        out_shape=jax.ShapeDtypeStruct((NPAD, D), jnp.float32),
    )(x, w)


def _combine_hedge_body(h0, h1, bd, o):
    deg = bd[...]
    inv = jnp.where(deg > 0, 1.0 / deg, 0.0)
    o[...] = (h0[...] + h1[...]) * inv


def _combine_hedge(h0, h1, bd):
    return pl.pallas_call(
        _combine_hedge_body,
        grid=(_GRID,),
        in_specs=[_feat_spec(), _feat_spec(), _col_spec()],
        out_specs=_feat_spec(),
        out_shape=jax.ShapeDtypeStruct((NPAD, D), jnp.float32),
    )(h0, h1, bd)


def _mid_body(q0, q1, dd, b1r, w2, a, o):
    deg = dd[...]
    inv = jnp.where(deg > 0, 1.0 / deg, 0.0)
    t = (q0[...] + q1[...]) * inv + b1r[...]
    av = a[0, 0]
    t = jnp.where(t >= 0, t, av * t)
    o[...] = jnp.dot(t, w2[...], preferred_element_type=jnp.float32)


def _mid(q0, q1, dd, b1r, w2, a):
    return pl.pallas_call(
        _mid_body,
        grid=(_GRID,),
        in_specs=[_feat_spec(), _feat_spec(), _col_spec(),
                  _fixed_spec((1, D)), _fixed_spec((D, D)),
                  _fixed_spec((1, 1))],
        out_specs=_feat_spec(),
        out_shape=jax.ShapeDtypeStruct((NPAD, D), jnp.float32),
    )(q0, q1, dd, b1r, w2, a)


def _final_body(q0, q1, dd, b2r, xr, a, o):
    deg = dd[...]
    inv = jnp.where(deg > 0, 1.0 / deg, 0.0)
    t = (q0[...] + q1[...]) * inv + b2r[...] + xr[...]
    av = a[0, 0]
    o[...] = jnp.where(t >= 0, t, av * t)


def _final(q0, q1, dd, b2r, xr, a):
    return pl.pallas_call(
        _final_body,
        grid=(_GRID,),
        in_specs=[_feat_spec(), _feat_spec(), _col_spec(),
                  _fixed_spec((1, D)), _feat_spec(), _fixed_spec((1, 1))],
        out_specs=_feat_spec(),
        out_shape=jax.ShapeDtypeStruct((NPAD, D), jnp.float32),
    )(q0, q1, dd, b2r, xr, a)


def kernel(x, edge_index, W1, b1, W2, b2, prelu_a):
    nidx = edge_index[0]
    hidx = edge_index[1]
    pad = N + (jnp.arange(EPAD - E, dtype=jnp.int32) % (NPAD - N))
    nidx_r = jnp.concatenate([nidx, pad]).reshape(TILES, CPT, CHUNK)
    hidx_r = jnp.concatenate([hidx, pad]).reshape(TILES, CPT, CHUNK)
    n2h = jnp.stack([nidx_r, hidx_r], axis=2)
    h2n = jnp.stack([hidx_r, nidx_r], axis=2)
    x_pad = jnp.pad(x, ((0, NPAD - N), (0, 0)))

    deg_idx = jnp.stack([nidx_r.reshape(NS, _DCPT, CHUNK),
                         hidx_r.reshape(NS, _DCPT, CHUNK)], axis=0)
    degs = _deg_kernel(deg_idx)
    dd = degs[0, :, 0:1]
    bd = degs[1, :, 0:1]

    b1r = b1.reshape(1, D)
    b2r = b2.reshape(1, D)
    a2 = jnp.asarray(prelu_a, jnp.float32).reshape(1, 1)

    xt1 = _matmul(x_pad, W1)
    hp = _gs_kernel(xt1, n2h)
    hf1 = _combine_hedge(hp[0], hp[1], bd)
    qp = _gs_kernel(hf1, h2n)
    xt2 = _mid(qp[0], qp[1], dd, b1r, W2, a2)
    hp2 = _gs_kernel(xt2, n2h)
    hf2 = _combine_hedge(hp2[0], hp2[1], bd)
    qp2 = _gs_kernel(hf2, h2n)
    out = _final(qp2[0], qp2[1], dd, b2r, x_pad, a2)
    return out[:N]
